# Initial kernel scaffold; baseline (speedup 1.0000x reference)
import jax
import jax.numpy as jnp
from jax.experimental import pallas as pl


def kernel(descriptors0, descriptors1):
    raise NotImplementedError("write your pallas kernel here")


# speedup vs baseline: 140.6337x; 140.6337x over previous
import functools

import jax
import jax.numpy as jnp
from jax import lax
from jax.experimental import pallas as pl
from jax.experimental.pallas import tpu as pltpu
from jax.experimental.pallas import tpu_sc as plsc

B, D, N, M = 4, 64, 4096, 4096
BM = 512

NUM_WORKERS = 32
LANES = 16
CHUNK = (B * N) // NUM_WORKERS
VECS = CHUNK // LANES
ROWS_PER_BATCH = N // CHUNK


def _argmax_block(at_ref, b_ref, idx_ref, val_ref):
    s = lax.dot_general(
        at_ref[0], b_ref[0],
        dimension_numbers=(((1,), (0,)), ((), ())),
        preferred_element_type=jnp.float32,
    )
    mx = jnp.max(s, axis=0)
    rows = lax.broadcasted_iota(jnp.int32, s.shape, 0)
    idx = jnp.min(jnp.where(s == mx[None, :], rows, jnp.int32(N)), axis=0)
    idx_ref[0, 0, :] = idx
    val_ref[0, 0, :] = (mx + 1.0) * 0.5


def _matmul_argmax(a, b):
    at = jnp.swapaxes(a, 1, 2)
    grid = (B, M // BM)
    idx, val = pl.pallas_call(
        _argmax_block,
        grid=grid,
        in_specs=[
            pl.BlockSpec((1, N, D), lambda bi, j: (bi, 0, 0)),
            pl.BlockSpec((1, D, BM), lambda bi, j: (bi, 0, j)),
        ],
        out_specs=[
            pl.BlockSpec((1, 1, BM), lambda bi, j: (bi, 0, j)),
            pl.BlockSpec((1, 1, BM), lambda bi, j: (bi, 0, j)),
        ],
        out_shape=[
            jax.ShapeDtypeStruct((B, 1, M), jnp.int32),
            jax.ShapeDtypeStruct((B, 1, M), jnp.float32),
        ],
        compiler_params=pltpu.CompilerParams(
            dimension_semantics=("parallel", "parallel"),
        ),
    )(at, b)
    return idx.reshape(B, M), val.reshape(B, M)


def _mutual_check_body(m0_hbm, m1_hbm, out_hbm, m1_v, m0_v, out_v):
    wid = lax.axis_index("s") * 2 + lax.axis_index("c")
    batch = wid // ROWS_PER_BATCH
    off = (wid % ROWS_PER_BATCH) * CHUNK
    pltpu.sync_copy(m1_hbm.at[batch], m1_v)
    pltpu.sync_copy(m0_hbm.at[pl.ds(wid * CHUNK, CHUNK)], m0_v)
    for i in range(VECS):
        idx = m0_v[pl.ds(i * LANES, LANES)]
        loop = plsc.load_gather(m1_v, [idx])
        inds = off + i * LANES + lax.iota(jnp.int32, LANES)
        out_v[pl.ds(i * LANES, LANES)] = jnp.where(loop == inds, idx, -1)
    pltpu.sync_copy(out_v, out_hbm.at[pl.ds(wid * CHUNK, CHUNK)])


def _mutual_check(m0, m1):
    run = pl.kernel(
        _mutual_check_body,
        mesh=plsc.VectorSubcoreMesh(core_axis_name="c", subcore_axis_name="s"),
        out_type=jax.ShapeDtypeStruct((B * N,), jnp.int32),
        scratch_types=[
            pltpu.VMEM((M,), jnp.int32),
            pltpu.VMEM((CHUNK,), jnp.int32),
            pltpu.VMEM((CHUNK,), jnp.int32),
        ],
        compiler_params=pltpu.CompilerParams(needs_layout_passes=False),
    )
    return run(m0.reshape(B * N), m1).reshape(B, N)


@jax.jit
def kernel(descriptors0, descriptors1):
    matches1, scores1 = _matmul_argmax(descriptors0, descriptors1)
    matches0, scores0 = _matmul_argmax(descriptors1, descriptors0)
    matches0 = _mutual_check(matches0, matches1)
    return matches0, matches1, scores0, scores1


# speedup vs baseline: 151.1823x; 1.0750x over previous
import functools

import jax
import jax.numpy as jnp
from jax import lax
from jax.experimental import pallas as pl
from jax.experimental.pallas import tpu as pltpu
from jax.experimental.pallas import tpu_sc as plsc

B, D, N, M = 4, 64, 4096, 4096
BM = 512

NUM_WORKERS = 32
LANES = 16
CHUNK = (B * N) // NUM_WORKERS
VECS = CHUNK // LANES
ROWS_PER_BATCH = N // CHUNK


def _argmax_block(at_ref, b_ref, idx_ref, val_ref):
    s = lax.dot_general(
        at_ref[0], b_ref[0],
        dimension_numbers=(((1,), (0,)), ((), ())),
        preferred_element_type=jnp.float32,
    )
    mx = jnp.max(s, axis=0)
    rows_f = lax.broadcasted_iota(jnp.int32, s.shape, 0).astype(jnp.float32)
    cand = jnp.where(s == mx[None, :], rows_f, jnp.float32(N))
    idx = jnp.min(cand, axis=0).astype(jnp.int32)
    idx_ref[0, 0, :] = idx
    val_ref[0, 0, :] = (mx + 1.0) * 0.5


def _matmul_argmax(a, b):
    at = jnp.swapaxes(a, 1, 2)
    grid = (B, M // BM)
    idx, val = pl.pallas_call(
        _argmax_block,
        grid=grid,
        in_specs=[
            pl.BlockSpec((1, N, D), lambda bi, j: (bi, 0, 0)),
            pl.BlockSpec((1, D, BM), lambda bi, j: (bi, 0, j)),
        ],
        out_specs=[
            pl.BlockSpec((1, 1, BM), lambda bi, j: (bi, 0, j)),
            pl.BlockSpec((1, 1, BM), lambda bi, j: (bi, 0, j)),
        ],
        out_shape=[
            jax.ShapeDtypeStruct((B, 1, M), jnp.int32),
            jax.ShapeDtypeStruct((B, 1, M), jnp.float32),
        ],
        compiler_params=pltpu.CompilerParams(
            dimension_semantics=("parallel", "parallel"),
        ),
    )(at, b)
    return idx.reshape(B, M), val.reshape(B, M)


def _mutual_check_body(m0_hbm, m1_hbm, out_hbm, m1_v, m0_v, out_v):
    wid = lax.axis_index("s") * 2 + lax.axis_index("c")
    batch = wid // ROWS_PER_BATCH
    off = (wid % ROWS_PER_BATCH) * CHUNK
    pltpu.sync_copy(m1_hbm.at[batch], m1_v)
    pltpu.sync_copy(m0_hbm.at[pl.ds(wid * CHUNK, CHUNK)], m0_v)
    for i in range(VECS):
        idx = m0_v[pl.ds(i * LANES, LANES)]
        loop = plsc.load_gather(m1_v, [idx])
        inds = off + i * LANES + lax.iota(jnp.int32, LANES)
        out_v[pl.ds(i * LANES, LANES)] = jnp.where(loop == inds, idx, -1)
    pltpu.sync_copy(out_v, out_hbm.at[pl.ds(wid * CHUNK, CHUNK)])


def _mutual_check(m0, m1):
    run = pl.kernel(
        _mutual_check_body,
        mesh=plsc.VectorSubcoreMesh(core_axis_name="c", subcore_axis_name="s"),
        out_type=jax.ShapeDtypeStruct((B * N,), jnp.int32),
        scratch_types=[
            pltpu.VMEM((M,), jnp.int32),
            pltpu.VMEM((CHUNK,), jnp.int32),
            pltpu.VMEM((CHUNK,), jnp.int32),
        ],
        compiler_params=pltpu.CompilerParams(needs_layout_passes=False),
    )
    return run(m0.reshape(B * N), m1).reshape(B, N)


@jax.jit
def kernel(descriptors0, descriptors1):
    matches1, scores1 = _matmul_argmax(descriptors0, descriptors1)
    matches0, scores0 = _matmul_argmax(descriptors1, descriptors0)
    matches0 = _mutual_check(matches0, matches1)
    return matches0, matches1, scores0, scores1


# speedup vs baseline: 177.9729x; 1.1772x over previous
import jax
import jax.numpy as jnp
from jax import lax
from jax.experimental import pallas as pl
from jax.experimental.pallas import tpu as pltpu
from jax.experimental.pallas import tpu_sc as plsc

B, D, N, M = 4, 64, 4096, 4096
BM = 2048

NUM_WORKERS = 32
LANES = 16
CHUNK = (B * N) // NUM_WORKERS
VECS = CHUNK // LANES
ROWS_PER_BATCH = N // CHUNK


def _argmax_block(at_ref, b_ref, idx_ref, val_ref):
    s = lax.dot_general(
        at_ref[0], b_ref[0],
        dimension_numbers=(((1,), (0,)), ((), ())),
        preferred_element_type=jnp.float32,
    )
    mx = jnp.max(s, axis=0)
    rows_f = lax.broadcasted_iota(jnp.int32, s.shape, 0).astype(jnp.float32)
    cand = jnp.where(s == mx[None, :], rows_f, jnp.float32(N))
    idx_ref[0, 0, :] = jnp.min(cand, axis=0).astype(jnp.int32)
    val_ref[0, 0, :] = (mx + 1.0) * 0.5


def _matmul_argmax(a, b):
    at = jnp.swapaxes(a, 1, 2)
    grid = (B, M // BM)
    idx, val = pl.pallas_call(
        _argmax_block,
        grid=grid,
        in_specs=[
            pl.BlockSpec((1, N, D), lambda bi, j: (bi, 0, 0)),
            pl.BlockSpec((1, D, BM), lambda bi, j: (bi, 0, j)),
        ],
        out_specs=[
            pl.BlockSpec((1, 1, BM), lambda bi, j: (bi, 0, j)),
            pl.BlockSpec((1, 1, BM), lambda bi, j: (bi, 0, j)),
        ],
        out_shape=[
            jax.ShapeDtypeStruct((B, 1, M), jnp.int32),
            jax.ShapeDtypeStruct((B, 1, M), jnp.float32),
        ],
        compiler_params=pltpu.CompilerParams(
            dimension_semantics=("parallel", "parallel"),
        ),
    )(at, b)
    return idx.reshape(B, M), val.reshape(B, M)


def _mutual_check_body(m0_hbm, m1_hbm, out_hbm, m1_v, m0_v, out_v):
    wid = lax.axis_index("s") * 2 + lax.axis_index("c")
    batch = wid // ROWS_PER_BATCH
    off = (wid % ROWS_PER_BATCH) * CHUNK
    pltpu.sync_copy(m1_hbm.at[batch], m1_v)
    pltpu.sync_copy(m0_hbm.at[batch, pl.ds(off, CHUNK)], m0_v)
    for i in range(VECS):
        idx = m0_v[pl.ds(i * LANES, LANES)]
        loop = plsc.load_gather(m1_v, [idx])
        inds = off + i * LANES + lax.iota(jnp.int32, LANES)
        out_v[pl.ds(i * LANES, LANES)] = jnp.where(loop == inds, idx, -1)
    pltpu.sync_copy(out_v, out_hbm.at[batch, pl.ds(off, CHUNK)])


def _mutual_check(m0, m1):
    run = pl.kernel(
        _mutual_check_body,
        mesh=plsc.VectorSubcoreMesh(core_axis_name="c", subcore_axis_name="s"),
        out_type=jax.ShapeDtypeStruct((B, N), jnp.int32),
        scratch_types=[
            pltpu.VMEM((M,), jnp.int32),
            pltpu.VMEM((CHUNK,), jnp.int32),
            pltpu.VMEM((CHUNK,), jnp.int32),
        ],
        compiler_params=pltpu.CompilerParams(needs_layout_passes=False),
    )
    return run(m0, m1)


@jax.jit
def kernel(descriptors0, descriptors1):
    matches1, scores1 = _matmul_argmax(descriptors0, descriptors1)
    matches0, scores0 = _matmul_argmax(descriptors1, descriptors0)
    matches0 = _mutual_check(matches0, matches1)
    return matches0, matches1, scores0, scores1


# speedup vs baseline: 193.3510x; 1.0864x over previous
import jax
import jax.numpy as jnp
from jax import lax
from jax.experimental import pallas as pl
from jax.experimental.pallas import tpu as pltpu
from jax.experimental.pallas import tpu_sc as plsc

B, D, N, M = 4, 64, 4096, 4096
BM = 2048

NUM_WORKERS = 32
LANES = 16
CHUNK = (B * N) // NUM_WORKERS
VECS = CHUNK // LANES
ROWS_PER_BATCH = N // CHUNK


def _argmax_block(at_ref, b_ref, idx_ref, val_ref):
    s = lax.dot_general(
        at_ref[0], b_ref[0],
        dimension_numbers=(((0,), (0,)), ((), ())),
        preferred_element_type=jnp.float32,
    )
    mx = jnp.max(s, axis=0)
    rows_f = lax.broadcasted_iota(jnp.int32, s.shape, 0).astype(jnp.float32)
    cand = jnp.where(s == mx[None, :], rows_f, jnp.float32(N))
    idx_ref[0, 0, :] = jnp.min(cand, axis=0).astype(jnp.int32)
    val_ref[0, 0, :] = (mx + 1.0) * 0.5


def _matmul_argmax(a, b):
    grid = (B, M // BM)
    idx, val = pl.pallas_call(
        _argmax_block,
        grid=grid,
        in_specs=[
            pl.BlockSpec((1, D, N), lambda bi, j: (bi, 0, 0)),
            pl.BlockSpec((1, D, BM), lambda bi, j: (bi, 0, j)),
        ],
        out_specs=[
            pl.BlockSpec((1, 1, BM), lambda bi, j: (bi, 0, j)),
            pl.BlockSpec((1, 1, BM), lambda bi, j: (bi, 0, j)),
        ],
        out_shape=[
            jax.ShapeDtypeStruct((B, 1, M), jnp.int32),
            jax.ShapeDtypeStruct((B, 1, M), jnp.float32),
        ],
        compiler_params=pltpu.CompilerParams(
            dimension_semantics=("parallel", "parallel"),
        ),
    )(a, b)
    return idx.reshape(B, M), val.reshape(B, M)


def _mutual_check_body(m0_hbm, m1_hbm, out_hbm, m1_v, m0_v, out_v):
    wid = lax.axis_index("s") * 2 + lax.axis_index("c")
    batch = wid // ROWS_PER_BATCH
    off = (wid % ROWS_PER_BATCH) * CHUNK
    pltpu.sync_copy(m1_hbm.at[batch], m1_v)
    pltpu.sync_copy(m0_hbm.at[batch, pl.ds(off, CHUNK)], m0_v)
    for i in range(VECS):
        idx = m0_v[pl.ds(i * LANES, LANES)]
        loop = plsc.load_gather(m1_v, [idx])
        inds = off + i * LANES + lax.iota(jnp.int32, LANES)
        out_v[pl.ds(i * LANES, LANES)] = jnp.where(loop == inds, idx, -1)
    pltpu.sync_copy(out_v, out_hbm.at[batch, pl.ds(off, CHUNK)])


def _mutual_check(m0, m1):
    run = pl.kernel(
        _mutual_check_body,
        mesh=plsc.VectorSubcoreMesh(core_axis_name="c", subcore_axis_name="s"),
        out_type=jax.ShapeDtypeStruct((B, N), jnp.int32),
        scratch_types=[
            pltpu.VMEM((M,), jnp.int32),
            pltpu.VMEM((CHUNK,), jnp.int32),
            pltpu.VMEM((CHUNK,), jnp.int32),
        ],
        compiler_params=pltpu.CompilerParams(needs_layout_passes=False),
    )
    return run(m0, m1)


@jax.jit
def kernel(descriptors0, descriptors1):
    matches1, scores1 = _matmul_argmax(descriptors0, descriptors1)
    matches0, scores0 = _matmul_argmax(descriptors1, descriptors0)
    matches0 = _mutual_check(matches0, matches1)
    return matches0, matches1, scores0, scores1


# speedup vs baseline: 195.4920x; 1.0111x over previous
import jax
import jax.numpy as jnp
from jax import lax
from jax.experimental import pallas as pl
from jax.experimental.pallas import tpu as pltpu
from jax.experimental.pallas import tpu_sc as plsc

B, D, N, M = 4, 64, 4096, 4096
BM = 2048

NUM_WORKERS = 32
LANES = 16
CHUNK = (B * N) // NUM_WORKERS
VECS = CHUNK // LANES
ROWS_PER_BATCH = N // CHUNK


def _argmax_block(at_ref, b_ref, idx_ref, val_ref):
    s = lax.dot_general(
        at_ref[0], b_ref[0],
        dimension_numbers=(((0,), (0,)), ((), ())),
        preferred_element_type=jnp.float32,
    )
    mx = jnp.max(s, axis=0)
    rows_f = lax.broadcasted_iota(jnp.int32, s.shape, 0).astype(jnp.float32)
    cand = jnp.where(s == mx[None, :], rows_f, jnp.float32(N))
    idx_ref[0, 0, :] = jnp.min(cand, axis=0).astype(jnp.int32)
    val_ref[0, 0, :] = (mx + 1.0) * 0.5


def _matmul_argmax(a, b):
    grid = (B, M // BM)
    idx, val = pl.pallas_call(
        _argmax_block,
        grid=grid,
        in_specs=[
            pl.BlockSpec((1, D, N), lambda bi, j: (bi, 0, 0)),
            pl.BlockSpec((1, D, BM), lambda bi, j: (bi, 0, j)),
        ],
        out_specs=[
            pl.BlockSpec((1, 1, BM), lambda bi, j: (bi, 0, j)),
            pl.BlockSpec((1, 1, BM), lambda bi, j: (bi, 0, j)),
        ],
        out_shape=[
            jax.ShapeDtypeStruct((B, 1, M), jnp.int32),
            jax.ShapeDtypeStruct((B, 1, M), jnp.float32),
        ],
        compiler_params=pltpu.CompilerParams(
            dimension_semantics=("parallel", "parallel"),
        ),
    )(a, b)
    return idx, val


def _mutual_check_body(m0_hbm, m1_hbm, out_hbm, m1_v, m0_v, out_v):
    wid = lax.axis_index("s") * 2 + lax.axis_index("c")
    batch = wid // ROWS_PER_BATCH
    off = (wid % ROWS_PER_BATCH) * CHUNK
    pltpu.sync_copy(m1_hbm.at[batch, 0], m1_v)
    pltpu.sync_copy(m0_hbm.at[batch, 0, pl.ds(off, CHUNK)], m0_v)
    for i in range(VECS):
        idx = m0_v[pl.ds(i * LANES, LANES)]
        loop = plsc.load_gather(m1_v, [idx])
        inds = off + i * LANES + lax.iota(jnp.int32, LANES)
        out_v[pl.ds(i * LANES, LANES)] = jnp.where(loop == inds, idx, -1)
    pltpu.sync_copy(out_v, out_hbm.at[batch, pl.ds(off, CHUNK)])


def _mutual_check(m0, m1):
    run = pl.kernel(
        _mutual_check_body,
        mesh=plsc.VectorSubcoreMesh(core_axis_name="c", subcore_axis_name="s"),
        out_type=jax.ShapeDtypeStruct((B, N), jnp.int32),
        scratch_types=[
            pltpu.VMEM((M,), jnp.int32),
            pltpu.VMEM((CHUNK,), jnp.int32),
            pltpu.VMEM((CHUNK,), jnp.int32),
        ],
        compiler_params=pltpu.CompilerParams(needs_layout_passes=False),
    )
    return run(m0, m1)


@jax.jit
def kernel(descriptors0, descriptors1):
    matches1_3d, scores1_3d = _matmul_argmax(descriptors0, descriptors1)
    matches0_3d, scores0_3d = _matmul_argmax(descriptors1, descriptors0)
    matches0 = _mutual_check(matches0_3d, matches1_3d)
    return (matches0, matches1_3d.reshape(B, M),
            scores0_3d.reshape(B, N), scores1_3d.reshape(B, M))
